# Initial kernel scaffold; baseline (speedup 1.0000x reference)
import jax
import jax.numpy as jnp
from jax.experimental import pallas as pl


def kernel(boxes, scores):
    raise NotImplementedError("write your pallas kernel here")


# speedup vs baseline: 33.7885x; 33.7885x over previous
import jax
import jax.numpy as jnp
from jax import lax
from jax.experimental import pallas as pl
from jax.experimental.pallas import tpu as pltpu

_CLS_THR = 0.05
_NMS_THR = 0.5
_PRE_NMS = 1000
_POST_NMS = 100
_P = 1024
_OUT_ROWS = 128
_BLK = 128


def _nms_body(data_ref, datat_ref, out_ref, s_ref):
    x1r = datat_ref[0:1, :]
    y1r = datat_ref[1:2, :]
    x2r = datat_ref[2:3, :]
    y2r = datat_ref[3:4, :]
    scr = datat_ref[4:5, :]
    area_r = jnp.maximum(x2r - x1r, 0.0) * jnp.maximum(y2r - y1r, 0.0)

    def build_block(b, _):
        rows = pl.ds(b * _BLK, _BLK)
        x1c = data_ref[rows, 0:1]
        y1c = data_ref[rows, 1:2]
        x2c = data_ref[rows, 2:3]
        y2c = data_ref[rows, 3:4]
        area_c = jnp.maximum(x2c - x1c, 0.0) * jnp.maximum(y2c - y1c, 0.0)
        iw = jnp.maximum(jnp.minimum(x2c, x2r) - jnp.maximum(x1c, x1r), 0.0)
        ih = jnp.maximum(jnp.minimum(y2c, y2r) - jnp.maximum(y1c, y1r), 0.0)
        inter = iw * ih
        union = area_c + area_r - inter + 1e-9
        gt = inter / union > _NMS_THR
        irow = jax.lax.broadcasted_iota(jnp.int32, (_BLK, _P), 0) + b * _BLK
        jcol = jax.lax.broadcasted_iota(jnp.int32, (_BLK, _P), 1)
        s_ref[rows, :] = jnp.where(gt & (irow < jcol), 1.0, 0.0)
        return 0

    lax.fori_loop(0, _P // _BLK, build_block, 0, unroll=True)

    keep0 = jnp.ones((1, _P), dtype=jnp.float32)

    def cond(carry):
        return carry[1]

    def body(carry):
        keep, _ = carry
        sup = jnp.dot(keep, s_ref[...], preferred_element_type=jnp.float32)
        new = jnp.where(sup >= 0.5, 0.0, 1.0)
        return new, jnp.any(new != keep)

    keep, _ = lax.while_loop(cond, body, (keep0, True))

    v = jnp.where((keep > 0.5) & (scr > _CLS_THR), 1.0, 0.0)

    irow2 = jax.lax.broadcasted_iota(jnp.int32, (_P, _P), 0)
    jcol2 = jax.lax.broadcasted_iota(jnp.int32, (_P, _P), 1)
    lt = jnp.where(irow2 <= jcol2, 1.0, 0.0)
    cum = jnp.dot(v, lt, preferred_element_type=jnp.float32)

    jcol3 = jax.lax.broadcasted_iota(jnp.int32, (_OUT_ROWS, 1), 0).astype(
        jnp.float32)
    m = jnp.where(cum <= jcol3, 1.0, 0.0)
    p = jnp.sum(m, axis=1, keepdims=True)
    icol = jax.lax.broadcasted_iota(jnp.int32, (_OUT_ROWS, _P), 1).astype(
        jnp.float32)
    g = jnp.where(icol == p, 1.0, 0.0)
    out_ref[...] = jnp.dot(g, data_ref[...],
                           preferred_element_type=jnp.float32)


def kernel(boxes, scores):
    top_scores, top_idx = lax.top_k(scores, _PRE_NMS)
    top_boxes = jnp.take(boxes, top_idx, axis=0)
    data = jnp.full((_P, 8), -1.0, dtype=jnp.float32)
    data = data.at[:, :4].set(0.0)
    data = data.at[:_PRE_NMS, :4].set(top_boxes)
    data = data.at[:_PRE_NMS, 4].set(top_scores)
    out = pl.pallas_call(
        _nms_body,
        out_shape=jax.ShapeDtypeStruct((_OUT_ROWS, 8), jnp.float32),
        scratch_shapes=[pltpu.VMEM((_P, _P), jnp.float32)],
    )(data, data.T)
    return out[:_POST_NMS, :5]
